# baseline (speedup 1.0000x reference)
import functools

import jax
import jax.numpy as jnp
from jax.experimental import pallas as pl
from jax.experimental.pallas import tpu as pltpu

N = 2048
_DOT = functools.partial(jnp.dot, preferred_element_type=jnp.float32)
_BF = jnp.bfloat16

_AB = 128
_NAB = N // _AB
_SB = 512
_NSB = N // _SB
_ENC_STEP = 3 * _NAB
_DEC_STEP = _ENC_STEP + 1
_GRAM0 = _DEC_STEP + 1
_NTILE = _NSB * _NSB
_NSTEPS = _GRAM0 + 3 * _NTILE


def _dot_nt(a, b):
    return jax.lax.dot_general(a, b, (((1,), (1,)), ((), ())),
                               preferred_element_type=jnp.float32)


def _gcn3(x, a, w1, w2, w3):
    u = _DOT(x, w1.astype(_BF)).astype(_BF)
    o = jnp.maximum(_DOT(a, u), 0.0).astype(_BF)
    o = jnp.maximum(_DOT(a, _DOT(o, w2.astype(_BF)).astype(_BF)), 0.0).astype(_BF)
    return jnp.maximum(_DOT(a, _DOT(o, w3.astype(_BF)).astype(_BF)), 0.0)


def _body(x_ref, xm_ref, a1_ref, a2_ref, g_ref,
          wge1_ref, wge2_ref, wge3_ref, wgd1_ref, wgd2_ref, wgd3_ref,
          whe1_ref, whe2_ref, whe3_ref, whd1_ref, whd2_ref, whd3_ref,
          alpha_ref,
          h_ref, s1_ref, s2_ref, s3_ref, x1_ref, x2_ref, x3_ref,
          a1s_ref, a2s_ref, gs_ref,
          hbf_ref, h1s_ref, h2s_ref, h3s_ref, x1s_ref, x2s_ref, x3s_ref):
    s = pl.program_id(0)

    for k, (src, dst) in enumerate(((a1_ref, a1s_ref), (a2_ref, a2s_ref),
                                    (g_ref, gs_ref))):
        base = k * _NAB

        @pl.when((s >= base) & (s < base + _NAB))
        def _(src=src, dst=dst, base=base):
            row = (s - base) * _AB
            dst[pl.ds(row, _AB), :] = src[:].astype(_BF)

    @pl.when(s == _ENC_STEP)
    def _():
        h1 = _gcn3(x_ref[:], a1s_ref[:], wge1_ref[:], wge2_ref[:], wge3_ref[:])
        h2 = _gcn3(xm_ref[:], a2s_ref[:], wge1_ref[:], wge2_ref[:], wge3_ref[:])
        h3 = _gcn3(x_ref[:], gs_ref[:], whe1_ref[:], whe2_ref[:], whe3_ref[:])
        alpha = alpha_ref[0, 0]
        h = alpha * 0.5 * (h1 + h2) + (1.0 - alpha) * h3
        h_ref[:] = h
        hbf_ref[:] = h.astype(_BF)
        h1s_ref[:] = h1.astype(_BF)
        h2s_ref[:] = h2.astype(_BF)
        h3s_ref[:] = h3.astype(_BF)

    @pl.when(s == _DEC_STEP)
    def _():
        h_bf = hbf_ref[:]
        x1 = _gcn3(h_bf, a1s_ref[:], wgd1_ref[:], wgd2_ref[:], wgd3_ref[:])
        x2 = _gcn3(h_bf, a2s_ref[:], wgd1_ref[:], wgd2_ref[:], wgd3_ref[:])
        x3 = _gcn3(h_bf, gs_ref[:], whd1_ref[:], whd2_ref[:], whd3_ref[:])
        x1_ref[:] = x1
        x2_ref[:] = x2
        x3_ref[:] = x3
        x1s_ref[:] = x1.astype(_BF)
        x2s_ref[:] = x2.astype(_BF)
        x3s_ref[:] = x3.astype(_BF)

    for k, (hs, xs, out) in enumerate(((h1s_ref, x1s_ref, s1_ref),
                                       (h2s_ref, x2s_ref, s2_ref),
                                       (h3s_ref, x3s_ref, s3_ref))):
        base = _GRAM0 + k * _NTILE

        @pl.when((s >= base) & (s < base + _NTILE))
        def _(hs=hs, xs=xs, out=out, base=base):
            t = s - base
            i = t // _NSB
            j = t % _NSB
            hi = hs[pl.ds(i * _SB, _SB), :]
            hj = hs[pl.ds(j * _SB, _SB), :]
            xi = xs[pl.ds(i * _SB, _SB), :]
            xj = xs[pl.ds(j * _SB, _SB), :]
            t_enc = jnp.tanh(0.5 * _dot_nt(hi, hj))
            t_dec = jnp.tanh(0.5 * _dot_nt(xi, xj))
            out[:] = 0.5 + 0.25 * (t_enc + t_dec)


def _gram_idx(base):
    def idx(s):
        t = jnp.clip(s - base, 0, _NTILE - 1)
        return (t // _NSB, t % _NSB)
    return idx


def kernel(x, x_mask, A1, A2, G, Wg_e1, Wg_e2, Wg_e3, Wg_d1, Wg_d2, Wg_d3,
           Wh_e1, Wh_e2, Wh_e3, Wh_d1, Wh_d2, Wh_d3, alpha):
    f32 = jnp.float32
    full = lambda shape: pl.BlockSpec(shape, lambda s: (0,) * len(shape))
    a_spec = lambda k: pl.BlockSpec(
        (_AB, N), lambda s, k=k: (jnp.clip(s - k * _NAB, 0, _NAB - 1), 0))
    w_specs = [full(w.shape) for w in
               (Wg_e1, Wg_e2, Wg_e3, Wg_d1, Wg_d2, Wg_d3,
                Wh_e1, Wh_e2, Wh_e3, Wh_d1, Wh_d2, Wh_d3)]
    out_shapes = (
        jax.ShapeDtypeStruct((N, 32), f32),
        jax.ShapeDtypeStruct((N, N), f32),
        jax.ShapeDtypeStruct((N, N), f32),
        jax.ShapeDtypeStruct((N, N), f32),
        jax.ShapeDtypeStruct((N, 256), f32),
        jax.ShapeDtypeStruct((N, 256), f32),
        jax.ShapeDtypeStruct((N, 256), f32),
    )
    out_specs = (
        full((N, 32)),
        pl.BlockSpec((_SB, _SB), _gram_idx(_GRAM0)),
        pl.BlockSpec((_SB, _SB), _gram_idx(_GRAM0 + _NTILE)),
        pl.BlockSpec((_SB, _SB), _gram_idx(_GRAM0 + 2 * _NTILE)),
        full((N, 256)),
        full((N, 256)),
        full((N, 256)),
    )
    scratch = [
        pltpu.VMEM((N, N), _BF),
        pltpu.VMEM((N, N), _BF),
        pltpu.VMEM((N, N), _BF),
        pltpu.VMEM((N, 32), _BF),
        pltpu.VMEM((N, 32), _BF),
        pltpu.VMEM((N, 32), _BF),
        pltpu.VMEM((N, 32), _BF),
        pltpu.VMEM((N, 256), _BF),
        pltpu.VMEM((N, 256), _BF),
        pltpu.VMEM((N, 256), _BF),
    ]
    return pl.pallas_call(
        _body,
        grid=(_NSTEPS,),
        in_specs=[full((N, 256)), full((N, 256)),
                  a_spec(0), a_spec(1), a_spec(2),
                  *w_specs, full((1, 1))],
        out_specs=out_specs,
        out_shape=out_shapes,
        scratch_shapes=scratch,
        compiler_params=pltpu.CompilerParams(
            vmem_limit_bytes=100 * 1024 * 1024),
    )(x.astype(_BF), x_mask.astype(_BF), A1, A2, G,
      Wg_e1, Wg_e2, Wg_e3, Wg_d1, Wg_d2, Wg_d3,
      Wh_e1, Wh_e2, Wh_e3, Wh_d1, Wh_d2, Wh_d3, alpha.reshape(1, 1))


# speedup vs baseline: 1.9205x; 1.9205x over previous
import functools

import jax
import jax.numpy as jnp
from jax.experimental import pallas as pl
from jax.experimental.pallas import tpu as pltpu

N = 2048
_DOT = functools.partial(jnp.dot, preferred_element_type=jnp.float32)
_BF = jnp.bfloat16

_AB = 128
_NAB = N // _AB
_SB = 512
_NSB = N // _SB
_ENC_STEP = 3 * _NAB
_DEC_STEP = _ENC_STEP + 1
_NSTEPS = _DEC_STEP + 1


def _dot_nt(a, b):
    return jax.lax.dot_general(a, b, (((1,), (1,)), ((), ())),
                               preferred_element_type=jnp.float32)


def _gcn3(x, a, w1, w2, w3):
    u = _DOT(x, w1.astype(_BF)).astype(_BF)
    o = jnp.maximum(_DOT(a, u), 0.0).astype(_BF)
    o = jnp.maximum(_DOT(a, _DOT(o, w2.astype(_BF)).astype(_BF)), 0.0).astype(_BF)
    return jnp.maximum(_DOT(a, _DOT(o, w3.astype(_BF)).astype(_BF)), 0.0)


def _body(x_ref, xm_ref, a1_ref, a2_ref, g_ref,
          wge1_ref, wge2_ref, wge3_ref, wgd1_ref, wgd2_ref, wgd3_ref,
          whe1_ref, whe2_ref, whe3_ref, whd1_ref, whd2_ref, whd3_ref,
          alpha_ref,
          h_ref, h1b_ref, h2b_ref, h3b_ref,
          x1_ref, x2_ref, x3_ref, x1b_ref, x2b_ref, x3b_ref,
          a1s_ref, a2s_ref, gs_ref, hbf_ref):
    s = pl.program_id(0)

    for k, (src, dst) in enumerate(((a1_ref, a1s_ref), (a2_ref, a2s_ref),
                                    (g_ref, gs_ref))):
        base = k * _NAB

        @pl.when((s >= base) & (s < base + _NAB))
        def _(src=src, dst=dst, base=base):
            row = (s - base) * _AB
            dst[pl.ds(row, _AB), :] = src[:].astype(_BF)

    @pl.when(s == _ENC_STEP)
    def _():
        h1 = _gcn3(x_ref[:], a1s_ref[:], wge1_ref[:], wge2_ref[:], wge3_ref[:])
        h2 = _gcn3(xm_ref[:], a2s_ref[:], wge1_ref[:], wge2_ref[:], wge3_ref[:])
        h3 = _gcn3(x_ref[:], gs_ref[:], whe1_ref[:], whe2_ref[:], whe3_ref[:])
        alpha = alpha_ref[0, 0]
        h = alpha * 0.5 * (h1 + h2) + (1.0 - alpha) * h3
        h_ref[:] = h
        hbf_ref[:] = h.astype(_BF)
        h1b_ref[:] = h1.astype(_BF)
        h2b_ref[:] = h2.astype(_BF)
        h3b_ref[:] = h3.astype(_BF)

    @pl.when(s == _DEC_STEP)
    def _():
        h_bf = hbf_ref[:]
        x1 = _gcn3(h_bf, a1s_ref[:], wgd1_ref[:], wgd2_ref[:], wgd3_ref[:])
        x2 = _gcn3(h_bf, a2s_ref[:], wgd1_ref[:], wgd2_ref[:], wgd3_ref[:])
        x3 = _gcn3(h_bf, gs_ref[:], whd1_ref[:], whd2_ref[:], whd3_ref[:])
        x1_ref[:] = x1
        x2_ref[:] = x2
        x3_ref[:] = x3
        x1b_ref[:] = x1.astype(_BF)
        x2b_ref[:] = x2.astype(_BF)
        x3b_ref[:] = x3.astype(_BF)

def _s_body(hi_ref, hj_ref, xi_ref, xj_ref, out_ref):
    t_enc = jnp.tanh(0.5 * _dot_nt(hi_ref[:], hj_ref[:]))
    t_dec = jnp.tanh(0.5 * _dot_nt(xi_ref[:], xj_ref[:]))
    out_ref[:] = 0.5 + 0.25 * (t_enc + t_dec)


def _s_avg(h_enc, x_dec):
    nb = N // _SB
    kh = h_enc.shape[1]
    kx = x_dec.shape[1]
    return pl.pallas_call(
        _s_body,
        grid=(nb, nb),
        in_specs=[
            pl.BlockSpec((_SB, kh), lambda i, j: (i, 0)),
            pl.BlockSpec((_SB, kh), lambda i, j: (j, 0)),
            pl.BlockSpec((_SB, kx), lambda i, j: (i, 0)),
            pl.BlockSpec((_SB, kx), lambda i, j: (j, 0)),
        ],
        out_specs=pl.BlockSpec((_SB, _SB), lambda i, j: (i, j)),
        out_shape=jax.ShapeDtypeStruct((N, N), jnp.float32),
    )(h_enc, h_enc, x_dec, x_dec)


def kernel(x, x_mask, A1, A2, G, Wg_e1, Wg_e2, Wg_e3, Wg_d1, Wg_d2, Wg_d3,
           Wh_e1, Wh_e2, Wh_e3, Wh_d1, Wh_d2, Wh_d3, alpha):
    f32 = jnp.float32
    full = lambda shape: pl.BlockSpec(shape, lambda s: (0,) * len(shape))
    a_spec = lambda k: pl.BlockSpec(
        (_AB, N), lambda s, k=k: (jnp.clip(s - k * _NAB, 0, _NAB - 1), 0))
    w_specs = [full(w.shape) for w in
               (Wg_e1, Wg_e2, Wg_e3, Wg_d1, Wg_d2, Wg_d3,
                Wh_e1, Wh_e2, Wh_e3, Wh_d1, Wh_d2, Wh_d3)]
    out_shapes = (
        jax.ShapeDtypeStruct((N, 32), f32),
        jax.ShapeDtypeStruct((N, 32), _BF),
        jax.ShapeDtypeStruct((N, 32), _BF),
        jax.ShapeDtypeStruct((N, 32), _BF),
        jax.ShapeDtypeStruct((N, 256), f32),
        jax.ShapeDtypeStruct((N, 256), f32),
        jax.ShapeDtypeStruct((N, 256), f32),
        jax.ShapeDtypeStruct((N, 256), _BF),
        jax.ShapeDtypeStruct((N, 256), _BF),
        jax.ShapeDtypeStruct((N, 256), _BF),
    )
    out_specs = (
        full((N, 32)),
        full((N, 32)), full((N, 32)), full((N, 32)),
        full((N, 256)), full((N, 256)), full((N, 256)),
        full((N, 256)), full((N, 256)), full((N, 256)),
    )
    scratch = [
        pltpu.VMEM((N, N), _BF),
        pltpu.VMEM((N, N), _BF),
        pltpu.VMEM((N, N), _BF),
        pltpu.VMEM((N, 32), _BF),
    ]
    h, h1b, h2b, h3b, x1, x2, x3, x1b, x2b, x3b = pl.pallas_call(
        _body,
        grid=(_NSTEPS,),
        in_specs=[full((N, 256)), full((N, 256)),
                  a_spec(0), a_spec(1), a_spec(2),
                  *w_specs, full((1, 1))],
        out_specs=out_specs,
        out_shape=out_shapes,
        scratch_shapes=scratch,
        compiler_params=pltpu.CompilerParams(
            vmem_limit_bytes=100 * 1024 * 1024),
    )(x.astype(_BF), x_mask.astype(_BF), A1, A2, G,
      Wg_e1, Wg_e2, Wg_e3, Wg_d1, Wg_d2, Wg_d3,
      Wh_e1, Wh_e2, Wh_e3, Wh_d1, Wh_d2, Wh_d3, alpha.reshape(1, 1))
    s1 = _s_avg(h1b, x1b)
    s2 = _s_avg(h2b, x2b)
    s3 = _s_avg(h3b, x3b)
    return (h, s1, s2, s3, x1, x2, x3)


# speedup vs baseline: 21.8232x; 11.3633x over previous
import functools

import jax
import jax.numpy as jnp
from jax.experimental import pallas as pl
from jax.experimental.pallas import tpu as pltpu

N = 2048
_DOT = functools.partial(jnp.dot, preferred_element_type=jnp.float32)
_BF = jnp.bfloat16

_CH = 128
_NCH = N // _CH
_SB = 512


def _dot_nt(a, b):
    return jax.lax.dot_general(a, b, (((1,), (1,)), ((), ())),
                               preferred_element_type=jnp.float32)


def _gcn3(x, a, w1, w2, w3):
    u = _DOT(x, w1.astype(_BF)).astype(_BF)
    o = jnp.maximum(_DOT(a, u), 0.0).astype(_BF)
    o = jnp.maximum(_DOT(a, _DOT(o, w2.astype(_BF)).astype(_BF)), 0.0).astype(_BF)
    return jnp.maximum(_DOT(a, _DOT(o, w3.astype(_BF)).astype(_BF)), 0.0)


def _body(x_ref, xm_ref, a1_ref, a2_ref, g_ref,
          wge1_ref, wge2_ref, wge3_ref, wgd1_ref, wgd2_ref, wgd3_ref,
          whe1_ref, whe2_ref, whe3_ref, whd1_ref, whd2_ref, whd3_ref,
          alpha_ref,
          h_ref, h1b_ref, h2b_ref, h3b_ref,
          x1_ref, x2_ref, x3_ref, x1b_ref, x2b_ref, x3b_ref,
          a1s_ref, a2s_ref, gs_ref, stage_ref, sems):
    plan = [(a1s_ref, a1_ref, c) for c in range(_NCH)] \
         + [(a2s_ref, a2_ref, c) for c in range(_NCH)] \
         + [(gs_ref, g_ref, c) for c in range(_NCH)]

    def start(t):
        _, src, c = plan[t]
        pltpu.make_async_copy(
            src.at[pl.ds(c * _CH, _CH), :],
            stage_ref.at[t % 2],
            sems.at[t % 2],
        ).start()

    start(0)
    for t in range(len(plan)):
        if t + 1 < len(plan):
            start(t + 1)
        dst, src, c = plan[t]
        pltpu.make_async_copy(
            src.at[pl.ds(c * _CH, _CH), :],
            stage_ref.at[t % 2],
            sems.at[t % 2],
        ).wait()
        dst[pl.ds(c * _CH, _CH), :] = stage_ref[t % 2].astype(_BF)

    a1 = a1s_ref[:]
    a2 = a2s_ref[:]
    g = gs_ref[:]
    h1 = _gcn3(x_ref[:], a1, wge1_ref[:], wge2_ref[:], wge3_ref[:])
    h2 = _gcn3(xm_ref[:], a2, wge1_ref[:], wge2_ref[:], wge3_ref[:])
    h3 = _gcn3(x_ref[:], g, whe1_ref[:], whe2_ref[:], whe3_ref[:])
    alpha = alpha_ref[0, 0]
    h = alpha * 0.5 * (h1 + h2) + (1.0 - alpha) * h3
    h_ref[:] = h
    h1b_ref[:] = h1.astype(_BF)
    h2b_ref[:] = h2.astype(_BF)
    h3b_ref[:] = h3.astype(_BF)

    h_bf = h.astype(_BF)
    x1 = _gcn3(h_bf, a1, wgd1_ref[:], wgd2_ref[:], wgd3_ref[:])
    x2 = _gcn3(h_bf, a2, wgd1_ref[:], wgd2_ref[:], wgd3_ref[:])
    x3 = _gcn3(h_bf, g, whd1_ref[:], whd2_ref[:], whd3_ref[:])
    x1_ref[:] = x1
    x2_ref[:] = x2
    x3_ref[:] = x3
    x1b_ref[:] = x1.astype(_BF)
    x2b_ref[:] = x2.astype(_BF)
    x3b_ref[:] = x3.astype(_BF)


def _s_body(hi_ref, hj_ref, xi_ref, xj_ref, out_ref):
    t_enc = jnp.tanh(0.5 * _dot_nt(hi_ref[:], hj_ref[:]))
    t_dec = jnp.tanh(0.5 * _dot_nt(xi_ref[:], xj_ref[:]))
    out_ref[:] = 0.5 + 0.25 * (t_enc + t_dec)


def _s_avg(h_enc, x_dec):
    nb = N // _SB
    kh = h_enc.shape[1]
    kx = x_dec.shape[1]
    return pl.pallas_call(
        _s_body,
        grid=(nb, nb),
        in_specs=[
            pl.BlockSpec((_SB, kh), lambda i, j: (i, 0)),
            pl.BlockSpec((_SB, kh), lambda i, j: (j, 0)),
            pl.BlockSpec((_SB, kx), lambda i, j: (i, 0)),
            pl.BlockSpec((_SB, kx), lambda i, j: (j, 0)),
        ],
        out_specs=pl.BlockSpec((_SB, _SB), lambda i, j: (i, j)),
        out_shape=jax.ShapeDtypeStruct((N, N), jnp.float32),
    )(h_enc, h_enc, x_dec, x_dec)


def kernel(x, x_mask, A1, A2, G, Wg_e1, Wg_e2, Wg_e3, Wg_d1, Wg_d2, Wg_d3,
           Wh_e1, Wh_e2, Wh_e3, Wh_d1, Wh_d2, Wh_d3, alpha):
    f32 = jnp.float32
    vspec = pl.BlockSpec(memory_space=pltpu.MemorySpace.VMEM)
    aspec = pl.BlockSpec(memory_space=pltpu.MemorySpace.HBM)
    out_shapes = (
        jax.ShapeDtypeStruct((N, 32), f32),
        jax.ShapeDtypeStruct((N, 32), _BF),
        jax.ShapeDtypeStruct((N, 32), _BF),
        jax.ShapeDtypeStruct((N, 32), _BF),
        jax.ShapeDtypeStruct((N, 256), f32),
        jax.ShapeDtypeStruct((N, 256), f32),
        jax.ShapeDtypeStruct((N, 256), f32),
        jax.ShapeDtypeStruct((N, 256), _BF),
        jax.ShapeDtypeStruct((N, 256), _BF),
        jax.ShapeDtypeStruct((N, 256), _BF),
    )
    scratch = [
        pltpu.VMEM((N, N), _BF),
        pltpu.VMEM((N, N), _BF),
        pltpu.VMEM((N, N), _BF),
        pltpu.VMEM((2, _CH, N), f32),
        pltpu.SemaphoreType.DMA((2,)),
    ]
    h, h1b, h2b, h3b, x1, x2, x3, x1b, x2b, x3b = pl.pallas_call(
        _body,
        in_specs=[vspec, vspec, aspec, aspec, aspec] + [vspec] * 13,
        out_specs=(vspec,) * 10,
        out_shape=out_shapes,
        scratch_shapes=scratch,
        compiler_params=pltpu.CompilerParams(
            vmem_limit_bytes=100 * 1024 * 1024),
    )(x.astype(_BF), x_mask.astype(_BF), A1, A2, G,
      Wg_e1, Wg_e2, Wg_e3, Wg_d1, Wg_d2, Wg_d3,
      Wh_e1, Wh_e2, Wh_e3, Wh_d1, Wh_d2, Wh_d3, alpha.reshape(1, 1))
    s1 = _s_avg(h1b, x1b)
    s2 = _s_avg(h2b, x2b)
    s3 = _s_avg(h3b, x3b)
    return (h, s1, s2, s3, x1, x2, x3)


# speedup vs baseline: 24.2292x; 1.1102x over previous
import functools

import jax
import jax.numpy as jnp
from jax.experimental import pallas as pl
from jax.experimental.pallas import tpu as pltpu

N = 2048
_DOT = functools.partial(jnp.dot, preferred_element_type=jnp.float32)
_BF = jnp.bfloat16

_CH = 128
_NCH = N // _CH
_SB = 512
_NSB = N // _SB


def _dot_nt(a, b):
    return jax.lax.dot_general(a, b, (((1,), (1,)), ((), ())),
                               preferred_element_type=jnp.float32)


def _gcn3(x, a_ref, w1, w2, w3):
    u = _DOT(x, w1.astype(_BF)).astype(_BF)
    o = jnp.maximum(_DOT(a_ref[:], u), 0.0).astype(_BF)
    o = jnp.maximum(_DOT(a_ref[:], _DOT(o, w2.astype(_BF)).astype(_BF)), 0.0).astype(_BF)
    return jnp.maximum(_DOT(a_ref[:], _DOT(o, w3.astype(_BF)).astype(_BF)), 0.0)


def _body(x_ref, xm_ref, a1_ref, a2_ref, g_ref,
          wge1_ref, wge2_ref, wge3_ref, wgd1_ref, wgd2_ref, wgd3_ref,
          whe1_ref, whe2_ref, whe3_ref, whd1_ref, whd2_ref, whd3_ref,
          alpha_ref,
          h_ref, s1_ref, s2_ref, s3_ref, x1_ref, x2_ref, x3_ref,
          a1s_ref, a2s_ref, gs_ref, stage_ref, sems,
          h1s_ref, h2s_ref, h3s_ref, x1s_ref, x2s_ref, x3s_ref,
          tile_ref, tsems):
    plan = [(a1s_ref, a1_ref, c) for c in range(_NCH)] \
         + [(a2s_ref, a2_ref, c) for c in range(_NCH)] \
         + [(gs_ref, g_ref, c) for c in range(_NCH)]

    def stream_copy(t):
        _, src, c = plan[t]
        return pltpu.make_async_copy(
            src.at[pl.ds(c * _CH, _CH), :],
            stage_ref.at[t % 2],
            sems.at[t % 2],
        )

    stream_copy(0).start()
    for t in range(len(plan)):
        if t + 1 < len(plan):
            stream_copy(t + 1).start()
        dst, _, c = plan[t]
        stream_copy(t).wait()
        dst[pl.ds(c * _CH, _CH), :] = stage_ref[t % 2].astype(_BF)

    h1 = _gcn3(x_ref[:], a1s_ref, wge1_ref[:], wge2_ref[:], wge3_ref[:])
    h2 = _gcn3(xm_ref[:], a2s_ref, wge1_ref[:], wge2_ref[:], wge3_ref[:])
    h3 = _gcn3(x_ref[:], gs_ref, whe1_ref[:], whe2_ref[:], whe3_ref[:])
    alpha = alpha_ref[0, 0]
    h = alpha * 0.5 * (h1 + h2) + (1.0 - alpha) * h3
    h_ref[:] = h
    h1s_ref[:] = h1.astype(_BF)
    h2s_ref[:] = h2.astype(_BF)
    h3s_ref[:] = h3.astype(_BF)

    h_bf = h.astype(_BF)
    x1 = _gcn3(h_bf, a1s_ref, wgd1_ref[:], wgd2_ref[:], wgd3_ref[:])
    x2 = _gcn3(h_bf, a2s_ref, wgd1_ref[:], wgd2_ref[:], wgd3_ref[:])
    x3 = _gcn3(h_bf, gs_ref, whd1_ref[:], whd2_ref[:], whd3_ref[:])
    x1_ref[:] = x1
    x2_ref[:] = x2
    x3_ref[:] = x3
    x1s_ref[:] = x1.astype(_BF)
    x2s_ref[:] = x2.astype(_BF)
    x3s_ref[:] = x3.astype(_BF)

    pend = [None, None]
    idx = 0
    for hs, xs, out in ((h1s_ref, x1s_ref, s1_ref),
                        (h2s_ref, x2s_ref, s2_ref),
                        (h3s_ref, x3s_ref, s3_ref)):
        for t in range(_NSB * _NSB):
            i, j = divmod(t, _NSB)
            slot = idx % 2
            if pend[slot] is not None:
                pend[slot].wait()
            hi = hs[pl.ds(i * _SB, _SB), :]
            hj = hs[pl.ds(j * _SB, _SB), :]
            xi = xs[pl.ds(i * _SB, _SB), :]
            xj = xs[pl.ds(j * _SB, _SB), :]
            t_enc = jnp.tanh(0.5 * _dot_nt(hi, hj))
            t_dec = jnp.tanh(0.5 * _dot_nt(xi, xj))
            tile_ref[slot] = 0.5 + 0.25 * (t_enc + t_dec)
            cp = pltpu.make_async_copy(
                tile_ref.at[slot],
                out.at[pl.ds(i * _SB, _SB), pl.ds(j * _SB, _SB)],
                tsems.at[slot],
            )
            cp.start()
            pend[slot] = cp
            idx += 1
    for cp in pend:
        if cp is not None:
            cp.wait()


def kernel(x, x_mask, A1, A2, G, Wg_e1, Wg_e2, Wg_e3, Wg_d1, Wg_d2, Wg_d3,
           Wh_e1, Wh_e2, Wh_e3, Wh_d1, Wh_d2, Wh_d3, alpha):
    f32 = jnp.float32
    vspec = pl.BlockSpec(memory_space=pltpu.MemorySpace.VMEM)
    aspec = pl.BlockSpec(memory_space=pltpu.MemorySpace.HBM)
    out_shapes = (
        jax.ShapeDtypeStruct((N, 32), f32),
        jax.ShapeDtypeStruct((N, N), f32),
        jax.ShapeDtypeStruct((N, N), f32),
        jax.ShapeDtypeStruct((N, N), f32),
        jax.ShapeDtypeStruct((N, 256), f32),
        jax.ShapeDtypeStruct((N, 256), f32),
        jax.ShapeDtypeStruct((N, 256), f32),
    )
    out_specs = (vspec, aspec, aspec, aspec, vspec, vspec, vspec)
    scratch = [
        pltpu.VMEM((N, N), _BF),
        pltpu.VMEM((N, N), _BF),
        pltpu.VMEM((N, N), _BF),
        pltpu.VMEM((2, _CH, N), f32),
        pltpu.SemaphoreType.DMA((2,)),
        pltpu.VMEM((N, 32), _BF),
        pltpu.VMEM((N, 32), _BF),
        pltpu.VMEM((N, 32), _BF),
        pltpu.VMEM((N, 256), _BF),
        pltpu.VMEM((N, 256), _BF),
        pltpu.VMEM((N, 256), _BF),
        pltpu.VMEM((2, _SB, _SB), f32),
        pltpu.SemaphoreType.DMA((2,)),
    ]
    return pl.pallas_call(
        _body,
        in_specs=[vspec, vspec, aspec, aspec, aspec] + [vspec] * 13,
        out_specs=out_specs,
        out_shape=out_shapes,
        scratch_shapes=scratch,
        compiler_params=pltpu.CompilerParams(
            vmem_limit_bytes=100 * 1024 * 1024),
    )(x.astype(_BF), x_mask.astype(_BF), A1, A2, G,
      Wg_e1, Wg_e2, Wg_e3, Wg_d1, Wg_d2, Wg_d3,
      Wh_e1, Wh_e2, Wh_e3, Wh_d1, Wh_d2, Wh_d3, alpha.reshape(1, 1))


# speedup vs baseline: 25.9556x; 1.0713x over previous
import functools

import jax
import jax.numpy as jnp
from jax.experimental import pallas as pl
from jax.experimental.pallas import tpu as pltpu

N = 2048
_DOT = functools.partial(jnp.dot, preferred_element_type=jnp.float32)
_BF = jnp.bfloat16

_CH = 256
_NCH = N // _CH
_SB = 512
_NSB = N // _SB


def _dot_nt(a, b):
    return jax.lax.dot_general(a, b, (((1,), (1,)), ((), ())),
                               preferred_element_type=jnp.float32)


def _gcn3(x, a_ref, w1, w2, w3):
    u = _DOT(x, w1.astype(_BF)).astype(_BF)
    o = jnp.maximum(_DOT(a_ref[:], u), 0.0).astype(_BF)
    o = jnp.maximum(_DOT(a_ref[:], _DOT(o, w2.astype(_BF)).astype(_BF)), 0.0).astype(_BF)
    return jnp.maximum(_DOT(a_ref[:], _DOT(o, w3.astype(_BF)).astype(_BF)), 0.0)


def _body(x_ref, xm_ref, a1_ref, a2_ref, g_ref,
          wge1_ref, wge2_ref, wge3_ref, wgd1_ref, wgd2_ref, wgd3_ref,
          whe1_ref, whe2_ref, whe3_ref, whd1_ref, whd2_ref, whd3_ref,
          alpha_ref,
          h_ref, s1_ref, s2_ref, s3_ref, x1_ref, x2_ref, x3_ref,
          a1s_ref, a2s_ref, gs_ref, stage_ref, sems,
          hpack_ref, x1s_ref, x2s_ref, x3s_ref,
          tile_ref, tsems):
    plan = [(a1s_ref, a1_ref, c) for c in range(_NCH)] \
         + [(a2s_ref, a2_ref, c) for c in range(_NCH)] \
         + [(gs_ref, g_ref, c) for c in range(_NCH)]

    def stream_copy(t):
        _, src, c = plan[t]
        return pltpu.make_async_copy(
            src.at[pl.ds(c * _CH, _CH), :],
            stage_ref.at[t % 2],
            sems.at[t % 2],
        )

    stream_copy(0).start()
    for t in range(len(plan)):
        if t + 1 < len(plan):
            stream_copy(t + 1).start()
        dst, _, c = plan[t]
        stream_copy(t).wait()
        dst[pl.ds(c * _CH, _CH), :] = stage_ref[t % 2].astype(_BF)

    h1 = _gcn3(x_ref[:], a1s_ref, wge1_ref[:], wge2_ref[:], wge3_ref[:])
    h2 = _gcn3(xm_ref[:], a2s_ref, wge1_ref[:], wge2_ref[:], wge3_ref[:])
    h3 = _gcn3(x_ref[:], gs_ref, whe1_ref[:], whe2_ref[:], whe3_ref[:])
    alpha = alpha_ref[0, 0]
    h = alpha * 0.5 * (h1 + h2) + (1.0 - alpha) * h3
    h_ref[:] = h
    hpack_ref[:, 0:32] = h1.astype(_BF)
    hpack_ref[:, 32:64] = h2.astype(_BF)
    hpack_ref[:, 64:96] = h3.astype(_BF)

    h_bf = h.astype(_BF)
    x1 = _gcn3(h_bf, a1s_ref, wgd1_ref[:], wgd2_ref[:], wgd3_ref[:])
    x2 = _gcn3(h_bf, a2s_ref, wgd1_ref[:], wgd2_ref[:], wgd3_ref[:])
    x3 = _gcn3(h_bf, gs_ref, whd1_ref[:], whd2_ref[:], whd3_ref[:])
    x1_ref[:] = x1
    x2_ref[:] = x2
    x3_ref[:] = x3
    x1s_ref[:] = x1.astype(_BF)
    x2s_ref[:] = x2.astype(_BF)
    x3s_ref[:] = x3.astype(_BF)

    pend = [None, None]
    idx = 0
    for k, (xs, out) in enumerate(((x1s_ref, s1_ref),
                                   (x2s_ref, s2_ref),
                                   (x3s_ref, s3_ref))):
        for t in range(_NSB * _NSB):
            i, j = divmod(t, _NSB)
            slot = idx % 2
            if pend[slot] is not None:
                pend[slot].wait()
            hi = hpack_ref[pl.ds(i * _SB, _SB), k * 32:(k + 1) * 32]
            hj = hpack_ref[pl.ds(j * _SB, _SB), k * 32:(k + 1) * 32]
            xi = xs[pl.ds(i * _SB, _SB), :]
            xj = xs[pl.ds(j * _SB, _SB), :]
            t_enc = jnp.tanh(0.5 * _dot_nt(hi, hj))
            t_dec = jnp.tanh(0.5 * _dot_nt(xi, xj))
            tile_ref[slot] = 0.5 + 0.25 * (t_enc + t_dec)
            cp = pltpu.make_async_copy(
                tile_ref.at[slot],
                out.at[pl.ds(i * _SB, _SB), pl.ds(j * _SB, _SB)],
                tsems.at[slot],
            )
            cp.start()
            pend[slot] = cp
            idx += 1
    for cp in pend:
        if cp is not None:
            cp.wait()


def kernel(x, x_mask, A1, A2, G, Wg_e1, Wg_e2, Wg_e3, Wg_d1, Wg_d2, Wg_d3,
           Wh_e1, Wh_e2, Wh_e3, Wh_d1, Wh_d2, Wh_d3, alpha):
    f32 = jnp.float32
    vspec = pl.BlockSpec(memory_space=pltpu.MemorySpace.VMEM)
    aspec = pl.BlockSpec(memory_space=pltpu.MemorySpace.HBM)
    out_shapes = (
        jax.ShapeDtypeStruct((N, 32), f32),
        jax.ShapeDtypeStruct((N, N), f32),
        jax.ShapeDtypeStruct((N, N), f32),
        jax.ShapeDtypeStruct((N, N), f32),
        jax.ShapeDtypeStruct((N, 256), f32),
        jax.ShapeDtypeStruct((N, 256), f32),
        jax.ShapeDtypeStruct((N, 256), f32),
    )
    out_specs = (vspec, aspec, aspec, aspec, vspec, vspec, vspec)
    scratch = [
        pltpu.VMEM((N, N), _BF),
        pltpu.VMEM((N, N), _BF),
        pltpu.VMEM((N, N), _BF),
        pltpu.VMEM((2, _CH, N), f32),
        pltpu.SemaphoreType.DMA((2,)),
        pltpu.VMEM((N, 96), _BF),
        pltpu.VMEM((N, 256), _BF),
        pltpu.VMEM((N, 256), _BF),
        pltpu.VMEM((N, 256), _BF),
        pltpu.VMEM((2, _SB, _SB), f32),
        pltpu.SemaphoreType.DMA((2,)),
    ]
    return pl.pallas_call(
        _body,
        in_specs=[vspec, vspec, aspec, aspec, aspec] + [vspec] * 13,
        out_specs=out_specs,
        out_shape=out_shapes,
        scratch_shapes=scratch,
        compiler_params=pltpu.CompilerParams(
            vmem_limit_bytes=100 * 1024 * 1024),
    )(x.astype(_BF), x_mask.astype(_BF), A1, A2, G,
      Wg_e1, Wg_e2, Wg_e3, Wg_d1, Wg_d2, Wg_d3,
      Wh_e1, Wh_e2, Wh_e3, Wh_d1, Wh_d2, Wh_d3, alpha.reshape(1, 1))
